# baseline (device time: 140877 ns/iter reference)
import jax
import jax.numpy as jnp
from jax import lax
from jax.experimental import pallas as pl
from jax.experimental.pallas import tpu as pltpu

N_DEV = 4

B, SQ, D = 2, 512, 1024
H_LOC, DH = 8, 128


def _ring_allreduce(p):
    m, n = p.shape

    def body(p_ref, out_ref, comm_ref, send_sems, recv_sems):
        my = lax.axis_index("i")
        left = (my - 1) % N_DEV
        right = (my + 1) % N_DEV

        barrier_sem = pltpu.get_barrier_semaphore()
        for nbr in (left, right):
            pl.semaphore_signal(
                barrier_sem, inc=1,
                device_id=(nbr,), device_id_type=pl.DeviceIdType.MESH,
            )
        pl.semaphore_wait(barrier_sem, 2)

        comm_ref[0] = p_ref[...]
        out_ref[...] = p_ref[...].astype(jnp.float32)

        for h in range(N_DEV - 1):
            rdma = pltpu.make_async_remote_copy(
                src_ref=comm_ref.at[h],
                dst_ref=comm_ref.at[h + 1],
                send_sem=send_sems.at[h],
                recv_sem=recv_sems.at[h],
                device_id=(right,),
                device_id_type=pl.DeviceIdType.MESH,
            )
            rdma.start()
            rdma.wait()
            out_ref[...] += comm_ref[h + 1].astype(jnp.float32)

    return pl.pallas_call(
        body,
        out_shape=jax.ShapeDtypeStruct((m, n), jnp.float32),
        in_specs=[pl.BlockSpec(memory_space=pltpu.VMEM)],
        out_specs=pl.BlockSpec(memory_space=pltpu.VMEM),
        scratch_shapes=[
            pltpu.VMEM((N_DEV, m, n), p.dtype),
            pltpu.SemaphoreType.DMA((N_DEV - 1,)),
            pltpu.SemaphoreType.DMA((N_DEV - 1,)),
        ],
        compiler_params=pltpu.CompilerParams(collective_id=0),
    )(p)


def kernel(x, Wq, Wk, Wv, Wo):
    bf16 = jnp.bfloat16
    x2 = x.reshape(B * SQ, D).astype(bf16)

    q = (x2 @ Wq.astype(bf16)).reshape(B, SQ, H_LOC, DH)
    k = (x2 @ Wk.astype(bf16)).reshape(B, SQ, H_LOC, DH)
    v = (x2 @ Wv.astype(bf16)).reshape(B, SQ, H_LOC, DH)

    inv = 1.0 / (10000.0 ** (jnp.arange(0, DH, 2, dtype=jnp.float32) / DH))
    pos = jnp.arange(SQ, dtype=jnp.float32)[:, None] * inv[None, :]
    cos = jnp.repeat(jnp.cos(pos), 2, axis=-1)[None, :, None, :]
    sin = jnp.repeat(jnp.sin(pos), 2, axis=-1)[None, :, None, :]

    def rot(t):
        tf = t.astype(jnp.float32)
        t2 = tf.reshape(B, SQ, H_LOC, DH // 2, 2)
        t_r = jnp.stack([-t2[..., 1], t2[..., 0]], axis=-1).reshape(
            B, SQ, H_LOC, DH
        )
        return (tf * cos + t_r * sin).astype(bf16)

    q = rot(q)
    k = rot(k)

    s = jnp.einsum(
        "bihd,bjhd->bhij", q, k, preferred_element_type=jnp.float32
    ) * 0.08838834764831843
    w = jax.nn.softmax(s, axis=-1).astype(bf16)
    ctx = jnp.einsum(
        "bhij,bjhd->bihd", w, v, preferred_element_type=jnp.float32
    ).astype(bf16)

    partial = ctx.reshape(B * SQ, H_LOC * DH) @ Wo.astype(bf16)

    out = _ring_allreduce(partial)
    return out.reshape(B, SQ, D)


# device time: 91218 ns/iter; 1.5444x vs baseline; 1.5444x over previous
import jax
import jax.numpy as jnp
from jax import lax
from jax.experimental import pallas as pl
from jax.experimental.pallas import tpu as pltpu

N_DEV = 4

B, SQ, D = 2, 512, 1024
H_LOC, DH = 8, 128
M = B * SQ
MC = M // N_DEV
SCALE = 0.08838834764831843


def _rope_tables():
    row = lax.broadcasted_iota(jnp.int32, (M, D), 0)
    lane = lax.broadcasted_iota(jnp.int32, (M, D), 1)
    pos = (row % SQ).astype(jnp.float32)
    d = lane % DH
    k = (d // 2).astype(jnp.float32)
    inv = jnp.exp(k * (-2.0 * jnp.log(10000.0) / DH))
    angle = pos * inv
    return jnp.cos(angle), jnp.sin(angle)


def _rot(t, cos, sin, even):
    tf = t.astype(jnp.float32)
    t_r = jnp.where(even, -jnp.roll(tf, -1, axis=1), jnp.roll(tf, 1, axis=1))
    return (tf * cos + t_r * sin).astype(jnp.bfloat16)


def kernel(x, Wq, Wk, Wv, Wo):
    x2 = x.reshape(M, D)

    def body(x_ref, wq_ref, wk_ref, wv_ref, wo_ref, out_ref,
             q_s, k_s, v_s, ctx_s, p_s, comm_ref, send_sems, recv_sems):
        my = lax.axis_index("i")
        left = lax.rem(my + N_DEV - 1, N_DEV)
        right = lax.rem(my + 1, N_DEV)

        barrier_sem = pltpu.get_barrier_semaphore()
        for nbr in (left, right):
            pl.semaphore_signal(
                barrier_sem, inc=1,
                device_id=(nbr,), device_id_type=pl.DeviceIdType.MESH,
            )
        pl.semaphore_wait(barrier_sem, 2)

        bf16 = jnp.bfloat16
        xb = x_ref[...].astype(bf16)

        cos, sin = _rope_tables()
        even = lax.broadcasted_iota(jnp.int32, (M, D), 1) % 2 == 0
        q_s[...] = _rot(
            jnp.dot(xb, wq_ref[...].astype(bf16),
                    preferred_element_type=jnp.float32),
            cos, sin, even)
        k_s[...] = _rot(
            jnp.dot(xb, wk_ref[...].astype(bf16),
                    preferred_element_type=jnp.float32),
            cos, sin, even)
        v_s[...] = jnp.dot(
            xb, wv_ref[...].astype(bf16),
            preferred_element_type=jnp.float32).astype(bf16)

        for b in range(B):
            r0 = b * SQ
            for h in range(H_LOC):
                c0 = h * DH
                q_bh = q_s[r0:r0 + SQ, c0:c0 + DH]
                k_bh = k_s[r0:r0 + SQ, c0:c0 + DH]
                v_bh = v_s[r0:r0 + SQ, c0:c0 + DH]
                s = lax.dot_general(
                    q_bh, k_bh, (((1,), (1,)), ((), ())),
                    preferred_element_type=jnp.float32) * SCALE
                s = s - jnp.max(s, axis=1, keepdims=True)
                e = jnp.exp(s)
                w = (e / jnp.sum(e, axis=1, keepdims=True)).astype(bf16)
                ctx_s[r0:r0 + SQ, c0:c0 + DH] = jnp.dot(
                    w, v_bh, preferred_element_type=jnp.float32).astype(bf16)

        p_s[...] = jnp.dot(
            ctx_s[...], wo_ref[...].astype(bf16),
            preferred_element_type=jnp.float32).astype(bf16)

        def chunk(s):
            return lax.rem(my - s + 2 * N_DEV, N_DEV) * MC

        def hop(src_slot, dst_slot, sem):
            return pltpu.make_async_remote_copy(
                src_ref=comm_ref.at[src_slot],
                dst_ref=comm_ref.at[dst_slot],
                send_sem=send_sems.at[sem],
                recv_sem=recv_sems.at[sem],
                device_id=(right,),
                device_id_type=pl.DeviceIdType.MESH,
            )

        comm_ref[0] = p_s[pl.ds(chunk(0), MC), :]
        for s in range(N_DEV - 1):
            if s > 0:
                comm_ref[s] += p_s[pl.ds(chunk(s), MC), :]
            rdma = hop(s, s + 1, s)
            rdma.start()
            rdma.wait()
        comm_ref[N_DEV - 1] += p_s[pl.ds(chunk(-1), MC), :]

        for t in range(N_DEV - 1):
            rdma = hop(N_DEV - 1 + t, N_DEV + t, N_DEV - 1 + t)
            rdma.start()
            rdma.wait()

        out_ref[pl.ds(chunk(-1), MC), :] = comm_ref[N_DEV - 1].astype(
            jnp.float32)
        for t in range(N_DEV - 1):
            out_ref[pl.ds(chunk(t), MC), :] = comm_ref[N_DEV + t].astype(
                jnp.float32)

    out = pl.pallas_call(
        body,
        out_shape=jax.ShapeDtypeStruct((M, D), jnp.float32),
        in_specs=[pl.BlockSpec(memory_space=pltpu.VMEM)] * 5,
        out_specs=pl.BlockSpec(memory_space=pltpu.VMEM),
        scratch_shapes=[
            pltpu.VMEM((M, D), jnp.bfloat16),
            pltpu.VMEM((M, D), jnp.bfloat16),
            pltpu.VMEM((M, D), jnp.bfloat16),
            pltpu.VMEM((M, D), jnp.bfloat16),
            pltpu.VMEM((M, D), jnp.bfloat16),
            pltpu.VMEM((2 * N_DEV - 1, MC, D), jnp.bfloat16),
            pltpu.SemaphoreType.DMA((2 * (N_DEV - 1),)),
            pltpu.SemaphoreType.DMA((2 * (N_DEV - 1),)),
        ],
        compiler_params=pltpu.CompilerParams(
            collective_id=0, vmem_limit_bytes=100 * 1024 * 1024
        ),
    )(x2, Wq, Wk, Wv, Wo)
    return out.reshape(B, SQ, D)


# device time: 85972 ns/iter; 1.6386x vs baseline; 1.0610x over previous
import jax
import jax.numpy as jnp
from jax import lax
from jax.experimental import pallas as pl
from jax.experimental.pallas import tpu as pltpu

N_DEV = 4

B, SQ, D = 2, 512, 1024
H_LOC, DH = 8, 128
M = B * SQ
MC = M // N_DEV
SCALE = 0.08838834764831843


def _rope_tables():
    row = lax.broadcasted_iota(jnp.int32, (M, D), 0)
    lane = lax.broadcasted_iota(jnp.int32, (M, D), 1)
    pos = (row % SQ).astype(jnp.float32)
    d = lane % DH
    k = (d // 2).astype(jnp.float32)
    inv = jnp.exp(k * (-2.0 * jnp.log(10000.0) / DH))
    angle = pos * inv
    return jnp.cos(angle), jnp.sin(angle)


def _rot(t, cos, sin, even):
    tf = t.astype(jnp.float32)
    t_r = jnp.where(even, -jnp.roll(tf, -1, axis=1), jnp.roll(tf, 1, axis=1))
    return (tf * cos + t_r * sin).astype(jnp.bfloat16)


def kernel(x, Wq, Wk, Wv, Wo):
    x2 = x.reshape(M, D)

    def body(x_ref, wq_ref, wk_ref, wv_ref, wo_ref, out_ref,
             q_s, k_s, v_s, ctx_blk, comm_ref, send_sems, recv_sems):
        my = lax.axis_index("i")
        left = lax.rem(my + N_DEV - 1, N_DEV)
        right = lax.rem(my + 1, N_DEV)

        barrier_sem = pltpu.get_barrier_semaphore()
        for nbr in (left, right):
            pl.semaphore_signal(
                barrier_sem, inc=1,
                device_id=(nbr,), device_id_type=pl.DeviceIdType.MESH,
            )
        pl.semaphore_wait(barrier_sem, 2)

        bf16 = jnp.bfloat16
        xb = x_ref[...].astype(bf16)

        cos, sin = _rope_tables()
        even = lax.broadcasted_iota(jnp.int32, (M, D), 1) % 2 == 0
        q_s[...] = _rot(
            jnp.dot(xb, wq_ref[...].astype(bf16),
                    preferred_element_type=jnp.float32),
            cos, sin, even)
        k_s[...] = _rot(
            jnp.dot(xb, wk_ref[...].astype(bf16),
                    preferred_element_type=jnp.float32),
            cos, sin, even)
        v_s[...] = jnp.dot(
            xb, wv_ref[...].astype(bf16),
            preferred_element_type=jnp.float32).astype(bf16)

        wo_b = wo_ref[...].astype(bf16)

        def chunk(s):
            return lax.rem(my - s + 2 * N_DEV, N_DEV) * MC

        def pchunk(s):
            roff = chunk(s)
            b0 = (roff // SQ) * SQ
            for h in range(H_LOC):
                c0 = h * DH
                q_b = q_s[pl.ds(roff, MC), c0:c0 + DH]
                k_b = k_s[pl.ds(b0, SQ), c0:c0 + DH]
                v_b = v_s[pl.ds(b0, SQ), c0:c0 + DH]
                sc = lax.dot_general(
                    q_b, k_b, (((1,), (1,)), ((), ())),
                    preferred_element_type=jnp.float32) * SCALE
                sc = sc - jnp.max(sc, axis=1, keepdims=True)
                e = jnp.exp(sc)
                w = (e / jnp.sum(e, axis=1, keepdims=True)).astype(bf16)
                ctx_blk[:, c0:c0 + DH] = jnp.dot(
                    w, v_b, preferred_element_type=jnp.float32).astype(bf16)
            return jnp.dot(
                ctx_blk[...], wo_b,
                preferred_element_type=jnp.float32).astype(bf16)

        def hop(src_slot, dst_slot, sem):
            return pltpu.make_async_remote_copy(
                src_ref=comm_ref.at[src_slot],
                dst_ref=comm_ref.at[dst_slot],
                send_sem=send_sems.at[sem],
                recv_sem=recv_sems.at[sem],
                device_id=(right,),
                device_id_type=pl.DeviceIdType.MESH,
            )

        comm_ref[0] = pchunk(0)
        for s in range(N_DEV - 1):
            rdma = hop(s, s + 1, s)
            rdma.start()
            p_next = pchunk(s + 1 if s < N_DEV - 2 else -1)
            rdma.wait()
            comm_ref[s + 1] += p_next

        for t in range(N_DEV - 1):
            rdma = hop(N_DEV - 1 + t, N_DEV + t, N_DEV - 1 + t)
            rdma.start()
            if t == 0:
                out_ref[pl.ds(chunk(-1), MC), :] = comm_ref[
                    N_DEV - 1].astype(jnp.float32)
            else:
                out_ref[pl.ds(chunk(t - 1), MC), :] = comm_ref[
                    N_DEV - 1 + t].astype(jnp.float32)
            rdma.wait()
        out_ref[pl.ds(chunk(N_DEV - 2), MC), :] = comm_ref[
            2 * N_DEV - 2].astype(jnp.float32)

    out = pl.pallas_call(
        body,
        out_shape=jax.ShapeDtypeStruct((M, D), jnp.float32),
        in_specs=[pl.BlockSpec(memory_space=pltpu.VMEM)] * 5,
        out_specs=pl.BlockSpec(memory_space=pltpu.VMEM),
        scratch_shapes=[
            pltpu.VMEM((M, D), jnp.bfloat16),
            pltpu.VMEM((M, D), jnp.bfloat16),
            pltpu.VMEM((M, D), jnp.bfloat16),
            pltpu.VMEM((MC, D), jnp.bfloat16),
            pltpu.VMEM((2 * N_DEV - 1, MC, D), jnp.bfloat16),
            pltpu.SemaphoreType.DMA((2 * (N_DEV - 1),)),
            pltpu.SemaphoreType.DMA((2 * (N_DEV - 1),)),
        ],
        compiler_params=pltpu.CompilerParams(
            collective_id=0, vmem_limit_bytes=100 * 1024 * 1024
        ),
    )(x2, Wq, Wk, Wv, Wo)
    return out.reshape(B, SQ, D)


# device time: 50574 ns/iter; 2.7856x vs baseline; 1.6999x over previous
import numpy as np

import jax
import jax.numpy as jnp
from jax import lax
from jax.experimental import pallas as pl
from jax.experimental.pallas import tpu as pltpu

N_DEV = 4

B, SQ, D = 2, 512, 1024
H_LOC, DH = 8, 128
M = B * SQ
MC = M // N_DEV
SCALE = 0.08838834764831843


def _host_rope_tables():
    inv = 1.0 / (10000.0 ** (np.arange(0, DH, 2) / DH))
    pos = np.arange(SQ)[:, None] * inv[None, :]
    cos = np.repeat(np.cos(pos), 2, axis=-1)
    sin = np.repeat(np.sin(pos), 2, axis=-1)
    even = (np.arange(DH) % 2 == 0)[None, :]
    s1 = np.where(even, -sin, 0.0)
    s2 = np.where(even, 0.0, sin)
    f = lambda a: jnp.asarray(a, dtype=jnp.bfloat16)
    return (f(cos * SCALE), f(s1 * SCALE), f(s2 * SCALE),
            f(cos), f(s1), f(s2))


def _rot(t, c, s1, s2):
    return t * c + jnp.roll(t, -1, axis=1) * s1 + jnp.roll(t, 1, axis=1) * s2


def kernel(x, Wq, Wk, Wv, Wo):
    x2 = x.reshape(M, D)
    tables = _host_rope_tables()

    D2 = D // 2

    def body(x_ref, wq_ref, wk_ref, wv_ref, wo_ref,
             cq_ref, s1q_ref, s2q_ref, ck_ref, s1k_ref, s2k_ref,
             out_ref, q_s, k_s, v_s, ctx_blk, snd, rcv, gA, gB, g2A, g2B,
             send_sems, recv_sems, sendG, recvG):
        my = lax.axis_index("i")
        left = lax.rem(my + N_DEV - 1, N_DEV)
        right = lax.rem(my + 1, N_DEV)
        diag = lax.rem(my + 2, N_DEV)

        barrier_sem = pltpu.get_barrier_semaphore()
        for nbr in (left, right, diag):
            pl.semaphore_signal(
                barrier_sem, inc=1,
                device_id=(nbr,), device_id_type=pl.DeviceIdType.MESH,
            )
        pl.semaphore_wait(barrier_sem, 3)

        bf16 = jnp.bfloat16
        xb = x_ref[...].astype(bf16)

        q_s[...] = jnp.dot(
            xb, wq_ref[...].astype(bf16),
            preferred_element_type=jnp.float32).astype(bf16)
        k_s[...] = jnp.dot(
            xb, wk_ref[...].astype(bf16),
            preferred_element_type=jnp.float32).astype(bf16)
        v_s[...] = jnp.dot(
            xb, wv_ref[...].astype(bf16),
            preferred_element_type=jnp.float32).astype(bf16)

        ck, s1k, s2k = ck_ref[...], s1k_ref[...], s2k_ref[...]
        for b in range(B):
            r0 = b * SQ
            for h in range(H_LOC):
                c0 = h * DH
                k_s[r0:r0 + SQ, c0:c0 + DH] = _rot(
                    k_s[r0:r0 + SQ, c0:c0 + DH], ck, s1k, s2k)

        wo_b = wo_ref[...].astype(bf16)

        def chunk(s):
            return lax.rem(my - s + 2 * N_DEV, N_DEV) * MC

        def pchunk(s):
            roff = chunk(s)
            b0 = (roff // SQ) * SQ
            p0 = roff - b0
            cq = cq_ref[pl.ds(p0, MC), :]
            s1q = s1q_ref[pl.ds(p0, MC), :]
            s2q = s2q_ref[pl.ds(p0, MC), :]
            for h in range(H_LOC):
                c0 = h * DH
                q_b = _rot(q_s[pl.ds(roff, MC), c0:c0 + DH], cq, s1q, s2q)
                k_b = k_s[pl.ds(b0, SQ), c0:c0 + DH]
                v_b = v_s[pl.ds(b0, SQ), c0:c0 + DH]
                sc = lax.dot_general(
                    q_b, k_b, (((1,), (1,)), ((), ())),
                    preferred_element_type=jnp.float32)
                e = jnp.exp(sc)
                recip = 1.0 / jnp.sum(e, axis=1, keepdims=True)
                ctx = lax.dot_general(
                    e.astype(bf16), v_b, (((1,), (0,)), ((), ())),
                    preferred_element_type=jnp.float32)
                ctx_blk[:, c0:c0 + DH] = (ctx * recip).astype(bf16)
            return jnp.dot(
                ctx_blk[...], wo_b,
                preferred_element_type=jnp.float32).astype(bf16)

        def push(slot, target):
            return pltpu.make_async_remote_copy(
                src_ref=snd.at[slot], dst_ref=rcv.at[slot],
                send_sem=send_sems.at[slot], recv_sem=recv_sems.at[slot],
                device_id=(target,), device_id_type=pl.DeviceIdType.MESH,
            )

        snd[2] = pchunk(1)
        r_diag = push(2, diag)
        r_diag.start()
        snd[0] = pchunk(0)
        r_left = push(0, left)
        r_left.start()
        snd[1] = pchunk(2)
        r_right = push(1, right)
        r_right.start()
        snd[3] = pchunk(-1)
        r_diag.wait()
        r_left.wait()
        r_right.wait()
        snd[3] += rcv[0] + (rcv[1] + rcv[2])

        r1 = pltpu.make_async_remote_copy(
            src_ref=snd.at[3], dst_ref=gA,
            send_sem=sendG.at[0], recv_sem=recvG.at[0],
            device_id=(right,), device_id_type=pl.DeviceIdType.MESH,
        )
        r2 = pltpu.make_async_remote_copy(
            src_ref=snd.at[3], dst_ref=gB,
            send_sem=sendG.at[1], recv_sem=recvG.at[1],
            device_id=(left,), device_id_type=pl.DeviceIdType.MESH,
        )
        r1.start()
        r2.start()
        out_ref[pl.ds(chunk(-1), MC), :] = snd[3]
        r1.wait()
        r2.wait()

        r3 = pltpu.make_async_remote_copy(
            src_ref=gA.at[:, pl.ds(0, D2)], dst_ref=g2A,
            send_sem=sendG.at[2], recv_sem=recvG.at[2],
            device_id=(right,), device_id_type=pl.DeviceIdType.MESH,
        )
        r4 = pltpu.make_async_remote_copy(
            src_ref=gB.at[:, pl.ds(D2, D2)], dst_ref=g2B,
            send_sem=sendG.at[3], recv_sem=recvG.at[3],
            device_id=(left,), device_id_type=pl.DeviceIdType.MESH,
        )
        r3.start()
        r4.start()
        out_ref[pl.ds(chunk(0), MC), :] = gA[...]
        out_ref[pl.ds(chunk(2), MC), :] = gB[...]
        r3.wait()
        r4.wait()
        out_ref[pl.ds(chunk(1), MC), :D2] = g2A[...]
        out_ref[pl.ds(chunk(1), MC), D2:] = g2B[...]

    out = pl.pallas_call(
        body,
        out_shape=jax.ShapeDtypeStruct((M, D), jnp.bfloat16),
        in_specs=[pl.BlockSpec(memory_space=pltpu.VMEM)] * 11,
        out_specs=pl.BlockSpec(memory_space=pltpu.VMEM),
        scratch_shapes=[
            pltpu.VMEM((M, D), jnp.bfloat16),
            pltpu.VMEM((M, D), jnp.bfloat16),
            pltpu.VMEM((M, D), jnp.bfloat16),
            pltpu.VMEM((MC, D), jnp.bfloat16),
            pltpu.VMEM((N_DEV, MC, D), jnp.bfloat16),
            pltpu.VMEM((N_DEV - 1, MC, D), jnp.bfloat16),
            pltpu.VMEM((MC, D), jnp.bfloat16),
            pltpu.VMEM((MC, D), jnp.bfloat16),
            pltpu.VMEM((MC, D // 2), jnp.bfloat16),
            pltpu.VMEM((MC, D // 2), jnp.bfloat16),
            pltpu.SemaphoreType.DMA((N_DEV - 1,)),
            pltpu.SemaphoreType.DMA((N_DEV - 1,)),
            pltpu.SemaphoreType.DMA((4,)),
            pltpu.SemaphoreType.DMA((4,)),
        ],
        compiler_params=pltpu.CompilerParams(
            collective_id=0, vmem_limit_bytes=100 * 1024 * 1024
        ),
    )(x2, Wq, Wk, Wv, Wo, *tables)
    return out.reshape(B, SQ, D)
